# Initial kernel scaffold; baseline (speedup 1.0000x reference)
import jax
import jax.numpy as jnp
from jax.experimental import pallas as pl


def kernel(pred_waveforms, obs_waveforms):
    raise NotImplementedError("write your pallas kernel here")


# speedup vs baseline: 2.4106x; 2.4106x over previous
import functools

import jax
import jax.numpy as jnp
from jax.experimental import pallas as pl

NT = 4096
NCOL = 384 * 3
BLK = 128
NBLK = NCOL // BLK


def _shift(x, j):
    if j > 0:
        return jnp.concatenate([x[j:], x[:j]], axis=0)
    j = -j
    return jnp.concatenate([x[NT - j:], x[:NT - j]], axis=0)


def _bitonic_substage(x, j, bitj, keep_min):
    p = jnp.where(bitj, _shift(x, -j), _shift(x, j))
    return jnp.where(keep_min, jnp.minimum(x, p), jnp.maximum(x, p))


def _wasserstein_kernel(pred_ref, obs_ref, out_ref):
    a = pred_ref[...]
    b = obs_ref[...]
    i = jax.lax.broadcasted_iota(jnp.int32, (NT, 1), 0)
    k = 2
    while k <= NT:
        kbit = (i & k) != 0
        j = k // 2
        while j >= 1:
            bitj = (i & j) != 0
            keep_min = bitj == kbit
            a = _bitonic_substage(a, j, bitj, keep_min)
            b = _bitonic_substage(b, j, bitj, keep_min)
            j //= 2
        k *= 2
    out_ref[0, 0, :] = jnp.sum(jnp.abs(a - b), axis=0)


@jax.jit
def kernel(pred_waveforms, obs_waveforms):
    pred = pred_waveforms.reshape(NT, NCOL)
    obs = obs_waveforms.reshape(NT, NCOL)
    partial = pl.pallas_call(
        _wasserstein_kernel,
        grid=(NBLK,),
        in_specs=[
            pl.BlockSpec((NT, BLK), lambda i: (0, i)),
            pl.BlockSpec((NT, BLK), lambda i: (0, i)),
        ],
        out_specs=pl.BlockSpec((1, 1, BLK), lambda i: (i, 0, 0)),
        out_shape=jax.ShapeDtypeStruct((NBLK, 1, BLK), jnp.float32),
    )(pred, obs)
    return jnp.sum(partial) / (NT * NCOL)


# speedup vs baseline: 2.7597x; 1.1448x over previous
import jax
import jax.numpy as jnp
from jax.experimental import pallas as pl

NT = 4096
NCOL = 384 * 3
BLK = 128
NBLK = NCOL // BLK


def _shift(x, j):
    m = x.shape[0]
    if j > 0:
        return jnp.concatenate([x[j:], x[:j]], axis=0)
    j = -j
    return jnp.concatenate([x[m - j:], x[:m - j]], axis=0)


def _uniform_large(x, j, min_down):
    m = x.shape[0]
    x4 = x.reshape(m // (2 * j), 2, j, BLK)
    a = x4[:, 0]
    b = x4[:, 1]
    lo = jnp.minimum(a, b)[:, None]
    hi = jnp.maximum(a, b)[:, None]
    pair = [lo, hi] if min_down else [hi, lo]
    return jnp.concatenate(pair, axis=1).reshape(m, BLK)


def _uniform_small(x, j, masks, min_down):
    bitj = masks[(x.shape[0], j)]
    u = _shift(x, j)
    d = _shift(x, -j)
    if min_down:
        return jnp.where(bitj, jnp.maximum(x, d), jnp.minimum(x, u))
    return jnp.where(bitj, jnp.minimum(x, d), jnp.maximum(x, u))


def _masked_substage(x, j, bitj, keep_min):
    p = jnp.where(bitj, _shift(x, -j), _shift(x, j))
    return jnp.where(keep_min, jnp.minimum(x, p), jnp.maximum(x, p))


def _stage_uniform(x, k, masks, min_down):
    j = k // 2
    while j >= 1:
        if j >= 8:
            x = _uniform_large(x, j, min_down)
        else:
            x = _uniform_small(x, j, masks, min_down)
        j //= 2
    return x


def _sort_axis0(x, masks, small_masks):
    i = small_masks
    k = 2
    while k <= 8:
        kbit = i[k]
        j = k // 2
        while j >= 1:
            bitj = i[j]
            x = _masked_substage(x, j, bitj, bitj == kbit)
            j //= 2
        k *= 2
    k = 16
    while k < NT:
        x6 = x.reshape(NT // (2 * k), 2, k, BLK)
        xa = x6[:, 0].reshape(NT // 2, BLK)
        xd = x6[:, 1].reshape(NT // 2, BLK)
        xa = _stage_uniform(xa, k, masks, True)
        xd = _stage_uniform(xd, k, masks, False)
        x = jnp.concatenate(
            [xa.reshape(NT // (2 * k), 1, k, BLK),
             xd.reshape(NT // (2 * k), 1, k, BLK)], axis=1).reshape(NT, BLK)
        k *= 2
    return _stage_uniform(x, NT, masks, True)


def _wasserstein_kernel(pred_ref, obs_ref, out_ref):
    ih = jax.lax.broadcasted_iota(jnp.int32, (NT // 2, 1), 0)
    i = jax.lax.broadcasted_iota(jnp.int32, (NT, 1), 0)
    masks = {(NT // 2, j): (ih & j) != 0 for j in (1, 2, 4)}
    masks.update({(NT, j): (i & j) != 0 for j in (1, 2, 4)})
    small_masks = {j: (i & j) != 0 for j in (1, 2, 4, 8)}
    a = _sort_axis0(pred_ref[...], masks, small_masks)
    b = _sort_axis0(obs_ref[...], masks, small_masks)
    out_ref[0, 0, :] = jnp.sum(jnp.abs(a - b), axis=0)


@jax.jit
def kernel(pred_waveforms, obs_waveforms):
    pred = pred_waveforms.reshape(NT, NCOL)
    obs = obs_waveforms.reshape(NT, NCOL)
    partial = pl.pallas_call(
        _wasserstein_kernel,
        grid=(NBLK,),
        in_specs=[
            pl.BlockSpec((NT, BLK), lambda i: (0, i)),
            pl.BlockSpec((NT, BLK), lambda i: (0, i)),
        ],
        out_specs=pl.BlockSpec((1, 1, BLK), lambda i: (i, 0, 0)),
        out_shape=jax.ShapeDtypeStruct((NBLK, 1, BLK), jnp.float32),
    )(pred, obs)
    return jnp.sum(partial) / (NT * NCOL)
